# baseline (speedup 1.0000x reference)
import functools

import jax
import jax.numpy as jnp
from jax import lax
from jax.experimental import pallas as pl
from jax.experimental.pallas import tpu as pltpu
from jax.experimental.pallas import tpu_sc as plsc

T = 2048
D = 2048
F = 1024
E = 8
K = 2
CAP = 640
NSLOT = E * CAP
FT = 512
NF = F // FT
TB = 512
NW = 32
_HI = jax.lax.Precision.HIGHEST


def _logits_body(x_ref, wr_ref, o_ref):
    o_ref[...] = lax.dot_general(
        x_ref[...], wr_ref[...], (((1,), (0,)), ((), ())),
        preferred_element_type=jnp.float32)


def _routing_body(lg_ref, tok_ref, g_ref, pos_ref):
    logits = lg_ref[...]
    iota_te = lax.broadcasted_iota(jnp.int32, (T, E), 1)
    v1 = jnp.max(logits, axis=1, keepdims=True)
    i1 = jnp.min(jnp.where(logits == v1, iota_te, E), axis=1, keepdims=True)
    masked = jnp.where(iota_te == i1, -1e30, logits)
    v2 = jnp.max(masked, axis=1, keepdims=True)
    i2 = jnp.min(jnp.where(masked == v2, iota_te, E), axis=1, keepdims=True)
    g1 = lax.logistic(v1 - v2)
    g2 = 1.0 - g1

    oh1 = (iota_te == i1).astype(jnp.float32)
    oh2 = (iota_te == i2).astype(jnp.float32)
    tot = oh1 + oh2
    tril = (lax.broadcasted_iota(jnp.int32, (T, T), 1)
            <= lax.broadcasted_iota(jnp.int32, (T, T), 0)).astype(jnp.bfloat16)
    csum = lax.dot_general(tril, tot.astype(jnp.bfloat16),
                           (((1,), (0,)), ((), ())),
                           preferred_element_type=jnp.float32)
    before = csum - tot
    r1 = jnp.sum(oh1 * before, axis=1, keepdims=True)
    r2 = jnp.sum(oh2 * (before + oh1), axis=1, keepdims=True)
    counts = csum[T - 1:T, :]

    capf = jnp.float32(CAP)
    cmin = jnp.min(counts, axis=1, keepdims=True)
    iota_e = lax.broadcasted_iota(jnp.int32, (1, E), 1)
    e_min = jnp.min(jnp.where(counts == cmin, iota_e, E), axis=1, keepdims=True)
    trash = e_min * CAP + (CAP - 1)

    r1i = (r1 + 0.5).astype(jnp.int32)
    r2i = (r2 + 0.5).astype(jnp.int32)
    p1 = jnp.where(r1 < capf, i1 * CAP + r1i, trash)
    p2 = jnp.where(r2 < capf, i2 * CAP + r2i, trash)
    pos_ref[0:T, :] = p1
    pos_ref[T:2 * T, :] = p2

    iota_cap = lax.broadcasted_iota(jnp.int32, (T, CAP), 1)
    tok_i = lax.broadcasted_iota(jnp.int32, (T, 1), 0)
    hif = (tok_i // 16).astype(jnp.float32)
    lof = (tok_i % 16).astype(jnp.float32)
    rhs1 = jnp.concatenate([hif, lof, g1], axis=1).astype(jnp.bfloat16)
    rhs2 = jnp.concatenate([hif, lof, g2], axis=1).astype(jnp.bfloat16)
    for e in range(E):
        ind1 = ((i1 == e) & (r1i == iota_cap)).astype(jnp.bfloat16)
        ind2 = ((i2 == e) & (r2i == iota_cap)).astype(jnp.bfloat16)
        sv = (lax.dot_general(ind1, rhs1, (((0,), (0,)), ((), ())),
                              preferred_element_type=jnp.float32)
              + lax.dot_general(ind2, rhs2, (((0,), (0,)), ((), ())),
                                preferred_element_type=jnp.float32))
        tok_ref[e * CAP:(e + 1) * CAP, :] = (
            16.0 * sv[:, 0:1] + sv[:, 1:2] + 0.5).astype(jnp.int32)
        g_ref[e * CAP:(e + 1) * CAP, :] = jnp.broadcast_to(sv[:, 2:3], (CAP, 128))


def _route(x, Wr):
    logits = pl.pallas_call(
        _logits_body,
        out_shape=jax.ShapeDtypeStruct((T, E), jnp.float32),
    )(x, Wr)
    return pl.pallas_call(
        _routing_body,
        out_shape=(
            jax.ShapeDtypeStruct((NSLOT, 1), jnp.int32),
            jax.ShapeDtypeStruct((NSLOT, 128), jnp.float32),
            jax.ShapeDtypeStruct((2 * T, 1), jnp.int32),
        ),
    )(logits)


def _sc_gather(src, idx, nrows, chunk):
    per_w = nrows // NW
    nchunk = per_w // chunk
    d = src.shape[1]
    idx2 = idx.reshape(NW, nchunk, chunk)
    NBUF = 4
    LA = 2

    @functools.partial(
        pl.kernel,
        mesh=plsc.VectorSubcoreMesh(core_axis_name="c", subcore_axis_name="s"),
        out_type=jax.ShapeDtypeStruct((nrows, d), jnp.float32),
        scratch_types=(
            [pltpu.VMEM((nchunk, chunk), jnp.int32)]
            + [pltpu.VMEM((chunk, d), jnp.float32)] * NBUF
            + [pltpu.SemaphoreType.DMA] * (2 * NBUF)
        ),
    )
    def k(src_hbm, idx_hbm, out_hbm, idx_v, *scr):
        bufs = scr[:NBUF]
        gsem = scr[NBUF:2 * NBUF]
        ssem = scr[2 * NBUF:]
        wid = lax.axis_index("s") * 2 + lax.axis_index("c")
        base = wid * per_w
        pltpu.sync_copy(idx_hbm.at[wid], idx_v)
        gathers = [None] * nchunk
        stores = [None] * nchunk

        def start_gather(j):
            gathers[j] = pltpu.async_copy(
                src_hbm.at[idx_v.at[j]], bufs[j % NBUF], gsem[j % NBUF])

        for j in range(min(LA, nchunk)):
            start_gather(j)
        for i in range(nchunk):
            gathers[i].wait()
            stores[i] = pltpu.async_copy(
                bufs[i % NBUF], out_hbm.at[pl.ds(base + i * chunk, chunk)],
                ssem[i % NBUF])
            j = i + LA
            if j < nchunk:
                if j >= NBUF:
                    stores[j - NBUF].wait()
                start_gather(j)
        for i in range(max(0, nchunk - NBUF), nchunk):
            stores[i].wait()

    return k(src, idx2)


def _gemm_body(xe_ref, wu_ref, wg_ref, wd_ref, g_ref, ye_ref):
    f = pl.program_id(1)
    xb = xe_ref[...].astype(jnp.bfloat16)
    up = lax.dot_general(xb, wu_ref[0].astype(jnp.bfloat16),
                         (((1,), (0,)), ((), ())),
                         preferred_element_type=jnp.float32)
    gt = lax.dot_general(xb, wg_ref[0].astype(jnp.bfloat16),
                         (((1,), (0,)), ((), ())),
                         preferred_element_type=jnp.float32)
    act = (up * gt * lax.logistic(gt)).astype(jnp.bfloat16)
    contrib = lax.dot_general(act, wd_ref[0].astype(jnp.bfloat16),
                              (((1,), (0,)), ((), ())),
                              preferred_element_type=jnp.float32)
    scaled = contrib * g_ref[...][:, 0:1]

    @pl.when(f == 0)
    def _():
        ye_ref[...] = scaled

    @pl.when(f > 0)
    def _():
        ye_ref[...] += scaled


def _gemm_half(xe_half, W_up_gate, W_down, g_col, e0, ye_prev=None):
    eh = E // 2
    body = _gemm_body if ye_prev is None else (
        lambda xe_ref, wu_ref, wg_ref, wd_ref, g_ref, prev_ref, ye_ref:
            _gemm_body(xe_ref, wu_ref, wg_ref, wd_ref, g_ref, ye_ref))
    in_specs = [
        pl.BlockSpec((CAP, D), lambda e, f: (e, 0)),
        pl.BlockSpec((1, D, FT), lambda e, f: (e + e0, 0, f)),
        pl.BlockSpec((1, D, FT), lambda e, f: (e + e0, 0, NF + f)),
        pl.BlockSpec((1, FT, D), lambda e, f: (e + e0, f, 0)),
        pl.BlockSpec((CAP, 128), lambda e, f: (e + e0, 0)),
    ]
    args = [xe_half, W_up_gate, W_up_gate, W_down, g_col]
    kwargs = {}
    if ye_prev is not None:
        in_specs.append(pl.BlockSpec(memory_space=pltpu.MemorySpace.HBM))
        args.append(ye_prev)
        kwargs["input_output_aliases"] = {5: 0}
    return pl.pallas_call(
        body,
        grid=(eh, NF),
        in_specs=in_specs,
        out_specs=pl.BlockSpec((CAP, D), lambda e, f: (e + e0, 0)),
        out_shape=jax.ShapeDtypeStruct((NSLOT, D), jnp.float32),
        **kwargs,
    )(*args)


def _combine_body(ya_ref, yb_ref, o_ref):
    o_ref[...] = ya_ref[...] + yb_ref[...]


def _combine(y12):
    nt = T // TB
    return pl.pallas_call(
        _combine_body,
        grid=(nt,),
        in_specs=[
            pl.BlockSpec((TB, D), lambda t: (t, 0)),
            pl.BlockSpec((TB, D), lambda t: (nt + t, 0)),
        ],
        out_specs=pl.BlockSpec((TB, D), lambda t: (t, 0)),
        out_shape=jax.ShapeDtypeStruct((T, D), jnp.float32),
    )(y12, y12)


def kernel(x, Wr, W_up_gate, W_down):
    tok_col, g_col, pos_col = _route(x, Wr)
    tok = tok_col.reshape(NSLOT)
    half = NSLOT // 2
    xe_a = _sc_gather(x, tok[:half], half, 8)
    xe_b = _sc_gather(x, tok[half:], half, 8)
    ye_a = _gemm_half(xe_a, W_up_gate, W_down, g_col, 0)
    ye = _gemm_half(xe_b, W_up_gate, W_down, g_col, E // 2, ye_prev=ye_a)
    y12 = _sc_gather(ye, pos_col.reshape(2 * T), 2 * T, 8)
    return _combine(y12)


# speedup vs baseline: 1.0154x; 1.0154x over previous
import functools

import jax
import jax.numpy as jnp
from jax import lax
from jax.experimental import pallas as pl
from jax.experimental.pallas import tpu as pltpu
from jax.experimental.pallas import tpu_sc as plsc

T = 2048
D = 2048
F = 1024
E = 8
K = 2
CAP = 640
NSLOT = E * CAP
FT = 512
NF = F // FT
TB = 512
NW = 32
_HI = jax.lax.Precision.HIGHEST


def _routing_body(x_ref, wr_ref, tok_ref, g_ref, pos_ref):
    logits = lax.dot_general(
        x_ref[...], wr_ref[...], (((1,), (0,)), ((), ())),
        preferred_element_type=jnp.float32)
    iota_te = lax.broadcasted_iota(jnp.int32, (T, E), 1)
    v1 = jnp.max(logits, axis=1, keepdims=True)
    i1 = jnp.min(jnp.where(logits == v1, iota_te, E), axis=1, keepdims=True)
    masked = jnp.where(iota_te == i1, -1e30, logits)
    v2 = jnp.max(masked, axis=1, keepdims=True)
    i2 = jnp.min(jnp.where(masked == v2, iota_te, E), axis=1, keepdims=True)
    g1 = lax.logistic(v1 - v2)
    g2 = 1.0 - g1

    oh1 = (iota_te == i1).astype(jnp.float32)
    oh2 = (iota_te == i2).astype(jnp.float32)
    tot = oh1 + oh2
    tril = (lax.broadcasted_iota(jnp.int32, (T, T), 1)
            <= lax.broadcasted_iota(jnp.int32, (T, T), 0)).astype(jnp.bfloat16)
    csum = lax.dot_general(tril, tot.astype(jnp.bfloat16),
                           (((1,), (0,)), ((), ())),
                           preferred_element_type=jnp.float32)
    before = csum - tot
    r1 = jnp.sum(oh1 * before, axis=1, keepdims=True)
    r2 = jnp.sum(oh2 * (before + oh1), axis=1, keepdims=True)
    counts = csum[T - 1:T, :]

    capf = jnp.float32(CAP)
    cmin = jnp.min(counts, axis=1, keepdims=True)
    iota_e = lax.broadcasted_iota(jnp.int32, (1, E), 1)
    e_min = jnp.min(jnp.where(counts == cmin, iota_e, E), axis=1, keepdims=True)
    trash = e_min * CAP + (CAP - 1)

    r1i = (r1 + 0.5).astype(jnp.int32)
    r2i = (r2 + 0.5).astype(jnp.int32)
    p1 = jnp.where(r1 < capf, i1 * CAP + r1i, trash)
    p2 = jnp.where(r2 < capf, i2 * CAP + r2i, trash)
    pos_ref[0:T, :] = p1
    pos_ref[T:2 * T, :] = p2

    iota_cap = lax.broadcasted_iota(jnp.int32, (T, CAP), 1)
    tok_i = lax.broadcasted_iota(jnp.int32, (T, 1), 0)
    hif = (tok_i // 16).astype(jnp.float32)
    lof = (tok_i % 16).astype(jnp.float32)
    rhs1 = jnp.concatenate([hif, lof, g1], axis=1).astype(jnp.bfloat16)
    rhs2 = jnp.concatenate([hif, lof, g2], axis=1).astype(jnp.bfloat16)
    for e in range(E):
        ind1 = ((i1 == e) & (r1i == iota_cap)).astype(jnp.bfloat16)
        ind2 = ((i2 == e) & (r2i == iota_cap)).astype(jnp.bfloat16)
        sv = (lax.dot_general(ind1, rhs1, (((0,), (0,)), ((), ())),
                              preferred_element_type=jnp.float32)
              + lax.dot_general(ind2, rhs2, (((0,), (0,)), ((), ())),
                                preferred_element_type=jnp.float32))
        tok_ref[e * CAP:(e + 1) * CAP, :] = (
            16.0 * sv[:, 0:1] + sv[:, 1:2] + 0.5).astype(jnp.int32)
        g_ref[e * CAP:(e + 1) * CAP, :] = jnp.broadcast_to(sv[:, 2:3], (CAP, 128))


def _route(x, Wr):
    return pl.pallas_call(
        _routing_body,
        out_shape=(
            jax.ShapeDtypeStruct((NSLOT, 1), jnp.int32),
            jax.ShapeDtypeStruct((NSLOT, 128), jnp.float32),
            jax.ShapeDtypeStruct((2 * T, 1), jnp.int32),
        ),
    )(x, Wr)


def _sc_gather(src, idx, nrows, chunk):
    per_w = nrows // NW
    nchunk = per_w // chunk
    d = src.shape[1]
    idx2 = idx.reshape(NW, nchunk, chunk)
    NBUF = 4
    LA = 2

    @functools.partial(
        pl.kernel,
        mesh=plsc.VectorSubcoreMesh(core_axis_name="c", subcore_axis_name="s"),
        out_type=jax.ShapeDtypeStruct((nrows, d), jnp.float32),
        scratch_types=(
            [pltpu.VMEM((nchunk, chunk), jnp.int32)]
            + [pltpu.VMEM((chunk, d), jnp.float32)] * NBUF
            + [pltpu.SemaphoreType.DMA] * (2 * NBUF)
        ),
    )
    def k(src_hbm, idx_hbm, out_hbm, idx_v, *scr):
        bufs = scr[:NBUF]
        gsem = scr[NBUF:2 * NBUF]
        ssem = scr[2 * NBUF:]
        wid = lax.axis_index("s") * 2 + lax.axis_index("c")
        base = wid * per_w
        pltpu.sync_copy(idx_hbm.at[wid], idx_v)
        gathers = [None] * nchunk
        stores = [None] * nchunk

        def start_gather(j):
            gathers[j] = pltpu.async_copy(
                src_hbm.at[idx_v.at[j]], bufs[j % NBUF], gsem[j % NBUF])

        for j in range(min(LA, nchunk)):
            start_gather(j)
        for i in range(nchunk):
            gathers[i].wait()
            stores[i] = pltpu.async_copy(
                bufs[i % NBUF], out_hbm.at[pl.ds(base + i * chunk, chunk)],
                ssem[i % NBUF])
            j = i + LA
            if j < nchunk:
                if j >= NBUF:
                    stores[j - NBUF].wait()
                start_gather(j)
        for i in range(max(0, nchunk - NBUF), nchunk):
            stores[i].wait()

    return k(src, idx2)


def _gemm_body(xe_ref, wu_ref, wg_ref, wd_ref, g_ref, ye_ref):
    f = pl.program_id(1)
    xb = xe_ref[...].astype(jnp.bfloat16)
    up = lax.dot_general(xb, wu_ref[0].astype(jnp.bfloat16),
                         (((1,), (0,)), ((), ())),
                         preferred_element_type=jnp.float32)
    gt = lax.dot_general(xb, wg_ref[0].astype(jnp.bfloat16),
                         (((1,), (0,)), ((), ())),
                         preferred_element_type=jnp.float32)
    act = (up * gt * lax.logistic(gt)).astype(jnp.bfloat16)
    contrib = lax.dot_general(act, wd_ref[0].astype(jnp.bfloat16),
                              (((1,), (0,)), ((), ())),
                              preferred_element_type=jnp.float32)
    scaled = contrib * g_ref[...][:, 0:1]

    @pl.when(f == 0)
    def _():
        ye_ref[...] = scaled

    @pl.when(f > 0)
    def _():
        ye_ref[...] += scaled


def _gemm(xe, W_up_gate, W_down, g_col):
    return pl.pallas_call(
        _gemm_body,
        grid=(E, NF),
        in_specs=[
            pl.BlockSpec((CAP, D), lambda e, f: (e, 0)),
            pl.BlockSpec((1, D, FT), lambda e, f: (e, 0, f)),
            pl.BlockSpec((1, D, FT), lambda e, f: (e, 0, NF + f)),
            pl.BlockSpec((1, FT, D), lambda e, f: (e, f, 0)),
            pl.BlockSpec((CAP, 128), lambda e, f: (e, 0)),
        ],
        out_specs=pl.BlockSpec((CAP, D), lambda e, f: (e, 0)),
        out_shape=jax.ShapeDtypeStruct((NSLOT, D), jnp.float32),
    )(xe, W_up_gate, W_up_gate, W_down, g_col)


def _combine_body(ya_ref, yb_ref, o_ref):
    o_ref[...] = ya_ref[...] + yb_ref[...]


def _combine(y12):
    nt = T // TB
    return pl.pallas_call(
        _combine_body,
        grid=(nt,),
        in_specs=[
            pl.BlockSpec((TB, D), lambda t: (t, 0)),
            pl.BlockSpec((TB, D), lambda t: (nt + t, 0)),
        ],
        out_specs=pl.BlockSpec((TB, D), lambda t: (t, 0)),
        out_shape=jax.ShapeDtypeStruct((T, D), jnp.float32),
    )(y12, y12)


def kernel(x, Wr, W_up_gate, W_down):
    tok_col, g_col, pos_col = _route(x, Wr)
    xe = _sc_gather(x, tok_col.reshape(NSLOT), NSLOT, 8)
    ye = _gemm(xe, W_up_gate, W_down, g_col)
    y12 = _sc_gather(ye, pos_col.reshape(2 * T), 2 * T, 8)
    return _combine(y12)
